# baseline (speedup 1.0000x reference)
import functools

import jax
import jax.numpy as jnp
from jax import lax
from jax.experimental import pallas as pl
from jax.experimental.pallas import tpu as pltpu
from jax.experimental.pallas import tpu_sc as plsc

S = 819200
D = 64

NC = 2
NS = 16
NW = NC * NS
B_W = S // NW
C = 128
NB = 4
N_CHUNK = B_W // C
N_GROUP = N_CHUNK // NB

_mesh = plsc.VectorSubcoreMesh(core_axis_name="c", subcore_axis_name="s")


@functools.partial(
    pl.kernel,
    mesh=_mesh,
    out_type=jax.ShapeDtypeStruct((S, D), jnp.float32),
    scratch_types=[
        pltpu.VMEM((B_W,), jnp.int32),
        pltpu.VMEM((NB * C, D), jnp.float32),
        pltpu.SemaphoreType.DMA,
        pltpu.SemaphoreType.DMA,
    ],
    compiler_params=pltpu.CompilerParams(use_tc_tiling_on_sc=False),
)
def _sc_gather(table, idx, out, idx_v, rows_v, gsem, ssem):
    wid = lax.axis_index("s") * NC + lax.axis_index("c")
    base = wid * B_W
    pltpu.sync_copy(idx.at[pl.ds(base, B_W)], idx_v)

    def group(g, carry):
        c0 = g * NB
        gathers = []
        for b in range(NB):
            gathers.append(
                pltpu.async_copy(
                    table.at[idx_v.at[pl.ds((c0 + b) * C, C)]],
                    rows_v.at[pl.ds(b * C, C)],
                    gsem,
                )
            )
        for cp in gathers:
            cp.wait()
        scatters = []
        for b in range(NB):
            scatters.append(
                pltpu.async_copy(
                    rows_v.at[pl.ds(b * C, C)],
                    out.at[pl.ds(base + (c0 + b) * C, C)],
                    ssem,
                )
            )
        for cp in scatters:
            cp.wait()
        return carry

    lax.fori_loop(0, N_GROUP, group, 0)


def kernel(flat_source, flat_idx):
    return _sc_gather(flat_source, flat_idx.astype(jnp.int32))


# speedup vs baseline: 1.0234x; 1.0234x over previous
import functools

import jax
import jax.numpy as jnp
from jax import lax
from jax.experimental import pallas as pl
from jax.experimental.pallas import tpu as pltpu
from jax.experimental.pallas import tpu_sc as plsc

S = 819200
D = 64

NC = 2
NS = 16
NW = NC * NS
B_W = S // NW
C = 256
NB = 2
N_CHUNK = B_W // C
G = N_CHUNK // NB
G2 = G // 2

_mesh = plsc.VectorSubcoreMesh(core_axis_name="c", subcore_axis_name="s")


@functools.partial(
    pl.kernel,
    mesh=_mesh,
    out_type=jax.ShapeDtypeStruct((S, D), jnp.float32),
    scratch_types=[
        pltpu.VMEM((B_W,), jnp.int32),
        pltpu.VMEM((NB * C, D), jnp.float32),
        pltpu.VMEM((NB * C, D), jnp.float32),
        pltpu.SemaphoreType.DMA,
        pltpu.SemaphoreType.DMA,
        pltpu.SemaphoreType.DMA,
        pltpu.SemaphoreType.DMA,
    ],
    compiler_params=pltpu.CompilerParams(use_tc_tiling_on_sc=False),
)
def _sc_gather(table, idx, out, idx_v, rows0, rows1, gsem0, gsem1, ssem0, ssem1):
    wid = lax.axis_index("s") * NC + lax.axis_index("c")
    base = wid * B_W
    pltpu.sync_copy(idx.at[pl.ds(base, B_W)], idx_v)

    def gather_cp(group, b, buf, sem):
        return pltpu.make_async_copy(
            table.at[idx_v.at[pl.ds((group * NB + b) * C, C)]],
            buf.at[pl.ds(b * C, C)],
            sem,
        )

    def scatter_cp(group, b, buf, sem):
        return pltpu.make_async_copy(
            buf.at[pl.ds(b * C, C)],
            out.at[pl.ds(base + (group * NB + b) * C, C)],
            sem,
        )

    for b in range(NB):
        gather_cp(0, b, rows0, gsem0).start()

    def body(gg, carry):
        e = 2 * gg
        o = e + 1
        for b in range(NB):
            gather_cp(e, b, rows0, gsem0).wait()
        for b in range(NB):
            gather_cp(o, b, rows1, gsem1).start()
        for b in range(NB):
            scatter_cp(e, b, rows0, ssem0).start()
        for b in range(NB):
            gather_cp(o, b, rows1, gsem1).wait()
        for b in range(NB):
            scatter_cp(e, b, rows0, ssem0).wait()

        @pl.when(gg + 1 < G2)
        def _():
            for b in range(NB):
                gather_cp(e + 2, b, rows0, gsem0).start()

        for b in range(NB):
            scatter_cp(o, b, rows1, ssem1).start()
        for b in range(NB):
            scatter_cp(o, b, rows1, ssem1).wait()
        return carry

    lax.fori_loop(0, G2, body, 0)


def kernel(flat_source, flat_idx):
    return _sc_gather(flat_source, flat_idx.astype(jnp.int32))
